# baseline (speedup 1.0000x reference)
import functools

import jax
import jax.numpy as jnp
from jax import lax
from jax.experimental import pallas as pl
from jax.experimental.pallas import tpu as pltpu
from jax.experimental.pallas import tpu_sc as plsc

N_NODES = 10000
N_PAD = 10240
E_TOT = 320000
D_FEAT = 128
HID = 128
ACT = 16
NC, NS = 2, 16
NW = NC * NS
CHUNK = 128
NCHT = E_TOT // CHUNK
NCHB = NCHT // NW
NEXTRA = NCHT - NCHB * NW
NPAIR = NCHB // 2 - 1
ROWS_PT = N_PAD // NS

_mesh = plsc.VectorSubcoreMesh(core_axis_name="c", subcore_axis_name="s")


@functools.partial(
    pl.kernel,
    out_type=jax.ShapeDtypeStruct((NC, N_PAD), jnp.float32),
    mesh=_mesh,
    scratch_types=[
        pltpu.VMEM((2, CHUNK), jnp.int32),
        pltpu.VMEM((2, CHUNK), jnp.int32),
        pltpu.VMEM((2, CHUNK), jnp.int32),
        pltpu.VMEM((2, CHUNK), jnp.int32),
        pltpu.VMEM((1, CHUNK), jnp.int32),
        pltpu.VMEM((CHUNK,), jnp.float32),
        pltpu.VMEM((ROWS_PT,), jnp.float32),
        pltpu.VMEM_SHARED((N_PAD,), jnp.float32),
        pltpu.SemaphoreType.DMA,
        pltpu.SemaphoreType.DMA,
        pltpu.SemaphoreType.DMA,
        pltpu.SemaphoreType.DMA,
    ],
)
def _deg_kernel(edge_hbm, deg_out, di0, di1, di2, di3, didx, ones_v,
                zero_v, deg_sh, sm0, sm1, sm2, sm3):
    c = lax.axis_index("c")
    s = lax.axis_index("s")
    wid = s * NC + c
    start = NCHB * wid + jnp.minimum(wid, NEXTRA)
    nch = NCHB + jnp.where(wid < NEXTRA, 1, 0)

    def esl(j, n):
        return edge_hbm.at[pl.ds(0, 2), pl.ds((start + j) * CHUNK, n)]

    def to_didx(eb):
        for l in range(CHUNK // 16):
            didx[0, pl.ds(l * 16, 16)] = eb[1, pl.ds(l * 16, 16)]

    for i in range(CHUNK // 16):
        ones_v[pl.ds(i * 16, 16)] = jnp.ones((16,), jnp.float32)
    for i in range(ROWS_PT // 16):
        zero_v[pl.ds(i * 16, 16)] = jnp.zeros((16,), jnp.float32)
    pltpu.sync_copy(zero_v, deg_sh.at[pl.ds(s * ROWS_PT, ROWS_PT)])
    plsc.subcore_barrier()

    bufs = (di0, di1, di2, di3)
    sems = (sm0, sm1, sm2, sm3)
    for k in range(4):
        pltpu.async_copy(esl(k, CHUNK), bufs[k], sems[k])

    def body(q, carry):
        for k in range(4):
            j = 4 * q + k
            pltpu.make_async_copy(esl(0, CHUNK), bufs[k], sems[k]).wait()
            to_didx(bufs[k])
            pltpu.sync_copy(ones_v, deg_sh.at[didx.at[0]], add=True)

            @pl.when(j + 4 < nch)
            def _():
                pltpu.async_copy(esl(j + 4, CHUNK), bufs[k], sems[k])
        return carry

    lax.fori_loop(0, NCHB // 4, body, 0)
    for k in range(2):
        pltpu.make_async_copy(esl(0, CHUNK), bufs[k], sems[k]).wait()
        to_didx(bufs[k])
        pltpu.sync_copy(ones_v, deg_sh.at[didx.at[0]], add=True)

    @pl.when(nch == NCHB + 1)
    def _():
        pltpu.make_async_copy(esl(0, CHUNK), bufs[2], sems[2]).wait()
        to_didx(bufs[2])
        pltpu.sync_copy(ones_v, deg_sh.at[didx.at[0]], add=True)

    plsc.subcore_barrier()
    pltpu.sync_copy(deg_sh.at[pl.ds(s * ROWS_PT, ROWS_PT)],
                    deg_out.at[c, pl.ds(s * ROWS_PT, ROWS_PT)])


@functools.partial(
    pl.kernel,
    out_type=[
        jax.ShapeDtypeStruct((N_PAD, HID), jnp.float32),
        jax.ShapeDtypeStruct((N_PAD, HID), jnp.float32),
    ],
    mesh=_mesh,
    scratch_types=[
        pltpu.VMEM((2, CHUNK), jnp.int32),
        pltpu.VMEM((2, CHUNK), jnp.int32),
        pltpu.VMEM((1, CHUNK), jnp.int32),
        pltpu.VMEM((1, CHUNK), jnp.int32),
        pltpu.VMEM((1, CHUNK), jnp.int32),
        pltpu.VMEM((1, CHUNK), jnp.int32),
        pltpu.VMEM((CHUNK, HID), jnp.float32),
        pltpu.VMEM((CHUNK, HID), jnp.float32),
        pltpu.VMEM_SHARED((N_PAD, HID), jnp.float32),
        pltpu.SemaphoreType.DMA,
        pltpu.SemaphoreType.DMA,
        pltpu.SemaphoreType.DMA,
        pltpu.SemaphoreType.DMA,
        pltpu.SemaphoreType.DMA,
        pltpu.SemaphoreType.DMA,
    ],
)
def _msg_kernel(xw_hbm, edge_hbm, acc0_out, acc1_out,
                ei0, ei1, sidx0, sidx1, didx0, didx1, buf0, buf1, acc_sh,
                sem0, sem1, semi0, semi1, sca0, sca1):
    c = lax.axis_index("c")
    s = lax.axis_index("s")
    wid = s * NC + c
    start = NCHB * wid + jnp.minimum(wid, NEXTRA)
    nch = NCHB + jnp.where(wid < NEXTRA, 1, 0)
    row0 = s * ROWS_PT

    def esl(j):
        return edge_hbm.at[pl.ds(0, 2), pl.ds((start + j) * CHUNK, CHUNK)]

    def unpack(eb, sb, db):
        for l in range(CHUNK // 16):
            sb[0, pl.ds(l * 16, 16)] = eb[0, pl.ds(l * 16, 16)]
            db[0, pl.ds(l * 16, 16)] = eb[1, pl.ds(l * 16, 16)]

    @pl.when((c == 0) & (s < NS - 1))
    def _():
        pltpu.sync_copy(xw_hbm.at[pl.ds(row0, ROWS_PT)],
                        acc_sh.at[pl.ds(row0, ROWS_PT)])

    @pl.when((c == 1) | (s == NS - 1))
    def _():
        def zb(i, carry):
            for l in range(HID // 16):
                buf0[i, pl.ds(l * 16, 16)] = jnp.zeros((16,), jnp.float32)
            return carry
        lax.fori_loop(0, CHUNK, zb, 0)

        @pl.when(c == 1)
        def _():
            def cp(i, carry):
                pltpu.sync_copy(buf0,
                                acc_sh.at[pl.ds(row0 + i * CHUNK, CHUNK)])
                return carry
            lax.fori_loop(0, ROWS_PT // CHUNK, cp, 0)

        @pl.when(c == 0)
        def _():
            pltpu.sync_copy(xw_hbm.at[pl.ds(row0, N_NODES - (NS - 1) * ROWS_PT)],
                            acc_sh.at[pl.ds(row0, N_NODES - (NS - 1) * ROWS_PT)])
            pltpu.sync_copy(buf0, acc_sh.at[pl.ds(N_NODES, CHUNK)])
            pltpu.sync_copy(buf0.at[pl.ds(0, N_PAD - N_NODES - CHUNK)],
                            acc_sh.at[pl.ds(N_NODES + CHUNK,
                                            N_PAD - N_NODES - CHUNK)])
    plsc.subcore_barrier()

    def gath(sb, bf, sm):
        pltpu.async_copy(xw_hbm.at[sb.at[0]], bf, sm)

    def gwait(sb, bf, sm):
        pltpu.make_async_copy(xw_hbm.at[sb.at[0]], bf, sm).wait()

    def scat(bf, db, sm):
        pltpu.async_copy(bf, acc_sh.at[db.at[0]], sm, add=True)

    def swait(bf, db, sm):
        pltpu.make_async_copy(bf, acc_sh.at[db.at[0]], sm).wait()

    pltpu.async_copy(esl(0), ei0, semi0)
    pltpu.make_async_copy(esl(0), ei0, semi0).wait()
    unpack(ei0, sidx0, didx0)
    gath(sidx0, buf0, sem0)
    pltpu.async_copy(esl(1), ei1, semi1)
    pltpu.make_async_copy(esl(0), ei1, semi1).wait()
    unpack(ei1, sidx1, didx1)
    gath(sidx1, buf1, sem1)
    gwait(sidx0, buf0, sem0)
    scat(buf0, didx0, sca0)
    pltpu.async_copy(esl(2), ei0, semi0)
    gwait(sidx1, buf1, sem1)
    scat(buf1, didx1, sca1)
    pltpu.async_copy(esl(3), ei1, semi1)
    pltpu.make_async_copy(esl(0), ei0, semi0).wait()
    swait(buf0, didx0, sca0)
    unpack(ei0, sidx0, didx0)
    gath(sidx0, buf0, sem0)

    def pair(p, carry):
        j0 = 2 * p
        pltpu.make_async_copy(esl(0), ei1, semi1).wait()
        swait(buf1, didx1, sca1)
        unpack(ei1, sidx1, didx1)
        gath(sidx1, buf1, sem1)
        gwait(sidx0, buf0, sem0)
        scat(buf0, didx0, sca0)
        pltpu.async_copy(esl(j0 + 2), ei0, semi0)
        gwait(sidx1, buf1, sem1)
        scat(buf1, didx1, sca1)
        pltpu.async_copy(esl(j0 + 3), ei1, semi1)
        pltpu.make_async_copy(esl(0), ei0, semi0).wait()
        swait(buf0, didx0, sca0)
        unpack(ei0, sidx0, didx0)
        gath(sidx0, buf0, sem0)
        return carry

    lax.fori_loop(1, NPAIR, pair, 0)
    pltpu.make_async_copy(esl(0), ei1, semi1).wait()
    swait(buf1, didx1, sca1)
    unpack(ei1, sidx1, didx1)
    gath(sidx1, buf1, sem1)
    gwait(sidx0, buf0, sem0)
    scat(buf0, didx0, sca0)

    @pl.when(nch == NCHB + 1)
    def _():
        pltpu.async_copy(esl(NCHB), ei0, semi0)
    gwait(sidx1, buf1, sem1)
    scat(buf1, didx1, sca1)

    @pl.when(nch == NCHB + 1)
    def _():
        pltpu.make_async_copy(esl(0), ei0, semi0).wait()
        swait(buf0, didx0, sca0)
        unpack(ei0, sidx0, didx0)
        gath(sidx0, buf0, sem0)
        gwait(sidx0, buf0, sem0)
        scat(buf0, didx0, sca0)

    swait(buf0, didx0, sca0)
    swait(buf1, didx1, sca1)

    plsc.subcore_barrier()

    @pl.when(c == 0)
    def _():
        pltpu.sync_copy(acc_sh.at[pl.ds(row0, ROWS_PT)],
                        acc0_out.at[pl.ds(row0, ROWS_PT)])

    @pl.when(c == 1)
    def _():
        pltpu.sync_copy(acc_sh.at[pl.ds(row0, ROWS_PT)],
                        acc1_out.at[pl.ds(row0, ROWS_PT)])


_BLKB = 1000
_GRIDB = N_NODES // _BLKB
_BLKD = 1000
_GRIDD = N_NODES // _BLKD


def _mm_body(x_ref, w_ref, out_ref):
    out_ref[...] = jnp.dot(x_ref[...], w_ref[...],
                           preferred_element_type=jnp.float32)


def _run_mm(x, w):
    return pl.pallas_call(
        _mm_body,
        grid=(_GRIDB,),
        in_specs=[
            pl.BlockSpec((_BLKB, D_FEAT), lambda i: (i, 0)),
            pl.BlockSpec((D_FEAT, HID), lambda i: (0, 0)),
        ],
        out_specs=pl.BlockSpec((_BLKB, HID), lambda i: (i, 0)),
        out_shape=jax.ShapeDtypeStruct((N_NODES, HID), jnp.float32),
    )(x, w)


def _scale_body(xw_ref, deg0_ref, deg1_ref, out_ref, dinv_ref):
    dinv = lax.rsqrt(deg0_ref[...] + deg1_ref[...] + 1.0)
    out_ref[...] = xw_ref[...] * dinv
    dinv_ref[...] = dinv


def _run_scale(xw_raw, deg0, deg1):
    return pl.pallas_call(
        _scale_body,
        grid=(_GRIDB,),
        in_specs=[
            pl.BlockSpec((_BLKB, HID), lambda i: (i, 0)),
            pl.BlockSpec((_BLKB, 1), lambda i: (i, 0)),
            pl.BlockSpec((_BLKB, 1), lambda i: (i, 0)),
        ],
        out_specs=[
            pl.BlockSpec((_BLKB, HID), lambda i: (i, 0)),
            pl.BlockSpec((_BLKB, 1), lambda i: (i, 0)),
        ],
        out_shape=[
            jax.ShapeDtypeStruct((N_NODES, HID), jnp.float32),
            jax.ShapeDtypeStruct((N_NODES, 1), jnp.float32),
        ],
    )(xw_raw, deg0, deg1)


def _head_body(acc0_ref, acc1_ref, dinv_ref, h_ref, bgcn_ref,
               wiht_ref, whht_ref, bih_ref, bhh_ref, wfc2t_ref, bfc2_ref,
               q_ref, hout_ref):
    dinv = dinv_ref[...]
    x = jnp.maximum(
        (acc0_ref[...] + acc1_ref[...]) * dinv + bgcn_ref[...], 0.0)
    h = h_ref[...]
    gi = jnp.dot(x, wiht_ref[...], preferred_element_type=jnp.float32) \
        + bih_ref[...]
    gh = jnp.dot(h, whht_ref[...], preferred_element_type=jnp.float32) \
        + bhh_ref[...]
    r = jax.nn.sigmoid(gi[:, :HID] + gh[:, :HID])
    z = jax.nn.sigmoid(gi[:, HID:2 * HID] + gh[:, HID:2 * HID])
    n = jnp.tanh(gi[:, 2 * HID:] + r * gh[:, 2 * HID:])
    hn = (1.0 - z) * n + z * h
    hout_ref[...] = hn
    q_ref[...] = jnp.dot(hn, wfc2t_ref[...],
                         preferred_element_type=jnp.float32) + bfc2_ref[...]


def _run_head(acc0, acc1, dinv, h, bgcn, wiht, whht, bih, bhh,
              wfc2t, bfc2):
    return pl.pallas_call(
        _head_body,
        grid=(_GRIDD,),
        in_specs=[
            pl.BlockSpec((_BLKD, HID), lambda i: (i, 0)),
            pl.BlockSpec((_BLKD, HID), lambda i: (i, 0)),
            pl.BlockSpec((_BLKD, 1), lambda i: (i, 0)),
            pl.BlockSpec((_BLKD, HID), lambda i: (i, 0)),
            pl.BlockSpec((1, HID), lambda i: (0, 0)),
            pl.BlockSpec((HID, 3 * HID), lambda i: (0, 0)),
            pl.BlockSpec((HID, 3 * HID), lambda i: (0, 0)),
            pl.BlockSpec((1, 3 * HID), lambda i: (0, 0)),
            pl.BlockSpec((1, 3 * HID), lambda i: (0, 0)),
            pl.BlockSpec((HID, ACT), lambda i: (0, 0)),
            pl.BlockSpec((1, ACT), lambda i: (0, 0)),
        ],
        out_specs=[
            pl.BlockSpec((_BLKD, ACT), lambda i: (i, 0)),
            pl.BlockSpec((_BLKD, HID), lambda i: (i, 0)),
        ],
        out_shape=[
            jax.ShapeDtypeStruct((N_NODES, ACT), jnp.float32),
            jax.ShapeDtypeStruct((N_NODES, HID), jnp.float32),
        ],
    )(acc0, acc1, dinv, h, bgcn, wiht, whht, bih, bhh, wfc2t, bfc2)


def kernel(inputs, hidden_state, edge_index, W_gcn, b_gcn,
           W_ih, W_hh, b_ih, b_hh, W_fc2, b_fc2):
    deg_parts = _deg_kernel(edge_index)
    xw_raw = _run_mm(inputs, W_gcn)
    deg0 = deg_parts[0, :N_NODES].reshape(N_NODES, 1)
    deg1 = deg_parts[1, :N_NODES].reshape(N_NODES, 1)
    xw, dinv = _run_scale(xw_raw, deg0, deg1)

    acc0, acc1 = _msg_kernel(xw, edge_index)

    q, h_new = _run_head(
        acc0, acc1, dinv, hidden_state.reshape(N_NODES, HID),
        b_gcn.reshape(1, HID), W_ih.T, W_hh.T,
        b_ih.reshape(1, 3 * HID), b_hh.reshape(1, 3 * HID),
        W_fc2.T, b_fc2.reshape(1, ACT))
    return (q, h_new)


# speedup vs baseline: 1.1932x; 1.1932x over previous
import functools

import jax
import jax.numpy as jnp
from jax import lax
from jax.experimental import pallas as pl
from jax.experimental.pallas import tpu as pltpu
from jax.experimental.pallas import tpu_sc as plsc

N_NODES = 10000
N_PAD = 10240
E_TOT = 320000
D_FEAT = 128
HID = 128
ACT = 16
NC, NS = 2, 16
NW = NC * NS
CHUNK = 128
NCHT = E_TOT // CHUNK
NCHB = NCHT // NW
NEXTRA = NCHT - NCHB * NW
NPAIR = NCHB // 2 - 1
ROWS_PT = N_PAD // NS

_mesh = plsc.VectorSubcoreMesh(core_axis_name="c", subcore_axis_name="s")


@functools.partial(
    pl.kernel,
    out_type=jax.ShapeDtypeStruct((NC, N_PAD), jnp.float32),
    mesh=_mesh,
    scratch_types=[
        pltpu.VMEM((2, CHUNK), jnp.int32),
        pltpu.VMEM((2, CHUNK), jnp.int32),
        pltpu.VMEM((2, CHUNK), jnp.int32),
        pltpu.VMEM((2, CHUNK), jnp.int32),
        pltpu.VMEM((1, CHUNK), jnp.int32),
        pltpu.VMEM((CHUNK,), jnp.float32),
        pltpu.VMEM((ROWS_PT,), jnp.float32),
        pltpu.VMEM_SHARED((N_PAD,), jnp.float32),
        pltpu.SemaphoreType.DMA,
        pltpu.SemaphoreType.DMA,
        pltpu.SemaphoreType.DMA,
        pltpu.SemaphoreType.DMA,
    ],
)
def _deg_kernel(edge_hbm, deg_out, di0, di1, di2, di3, didx, ones_v,
                zero_v, deg_sh, sm0, sm1, sm2, sm3):
    c = lax.axis_index("c")
    s = lax.axis_index("s")
    wid = s * NC + c
    start = NCHB * wid + jnp.minimum(wid, NEXTRA)
    nch = NCHB + jnp.where(wid < NEXTRA, 1, 0)

    def esl(j, n):
        return edge_hbm.at[pl.ds(0, 2), pl.ds((start + j) * CHUNK, n)]

    def to_didx(eb):
        for l in range(CHUNK // 16):
            didx[0, pl.ds(l * 16, 16)] = eb[1, pl.ds(l * 16, 16)]

    for i in range(CHUNK // 16):
        ones_v[pl.ds(i * 16, 16)] = jnp.ones((16,), jnp.float32)
    for i in range(ROWS_PT // 16):
        zero_v[pl.ds(i * 16, 16)] = jnp.zeros((16,), jnp.float32)
    pltpu.sync_copy(zero_v, deg_sh.at[pl.ds(s * ROWS_PT, ROWS_PT)])
    plsc.subcore_barrier()

    bufs = (di0, di1, di2, di3)
    sems = (sm0, sm1, sm2, sm3)
    for k in range(4):
        pltpu.async_copy(esl(k, CHUNK), bufs[k], sems[k])

    def body(q, carry):
        for k in range(4):
            j = 4 * q + k
            pltpu.make_async_copy(esl(0, CHUNK), bufs[k], sems[k]).wait()
            to_didx(bufs[k])
            pltpu.sync_copy(ones_v, deg_sh.at[didx.at[0]], add=True)

            @pl.when(j + 4 < nch)
            def _():
                pltpu.async_copy(esl(j + 4, CHUNK), bufs[k], sems[k])
        return carry

    lax.fori_loop(0, NCHB // 4, body, 0)
    for k in range(2):
        pltpu.make_async_copy(esl(0, CHUNK), bufs[k], sems[k]).wait()
        to_didx(bufs[k])
        pltpu.sync_copy(ones_v, deg_sh.at[didx.at[0]], add=True)

    @pl.when(nch == NCHB + 1)
    def _():
        pltpu.make_async_copy(esl(0, CHUNK), bufs[2], sems[2]).wait()
        to_didx(bufs[2])
        pltpu.sync_copy(ones_v, deg_sh.at[didx.at[0]], add=True)

    plsc.subcore_barrier()
    pltpu.sync_copy(deg_sh.at[pl.ds(s * ROWS_PT, ROWS_PT)],
                    deg_out.at[c, pl.ds(s * ROWS_PT, ROWS_PT)])


@functools.partial(
    pl.kernel,
    out_type=[
        jax.ShapeDtypeStruct((N_PAD, HID), jnp.float32),
        jax.ShapeDtypeStruct((N_PAD, HID), jnp.float32),
    ],
    mesh=_mesh,
    scratch_types=[
        pltpu.VMEM((2, CHUNK), jnp.int32),
        pltpu.VMEM((2, CHUNK), jnp.int32),
        pltpu.VMEM((1, CHUNK), jnp.int32),
        pltpu.VMEM((1, CHUNK), jnp.int32),
        pltpu.VMEM((1, CHUNK), jnp.int32),
        pltpu.VMEM((1, CHUNK), jnp.int32),
        pltpu.VMEM((CHUNK, HID), jnp.float32),
        pltpu.VMEM((CHUNK, HID), jnp.float32),
        pltpu.VMEM_SHARED((N_PAD, HID), jnp.float32),
        pltpu.SemaphoreType.DMA,
        pltpu.SemaphoreType.DMA,
        pltpu.SemaphoreType.DMA,
        pltpu.SemaphoreType.DMA,
    ],
)
def _msg_kernel(xw_hbm, edge_hbm, acc0_out, acc1_out,
                ei0, ei1, sidx0, sidx1, didx0, didx1, buf0, buf1, acc_sh,
                sem0, sem1, semi0, semi1):
    c = lax.axis_index("c")
    s = lax.axis_index("s")
    wid = s * NC + c
    start = NCHB * wid + jnp.minimum(wid, NEXTRA)
    nch = NCHB + jnp.where(wid < NEXTRA, 1, 0)
    row0 = s * ROWS_PT

    def esl(j):
        return edge_hbm.at[pl.ds(0, 2), pl.ds((start + j) * CHUNK, CHUNK)]

    def unpack(eb, sb, db):
        for l in range(CHUNK // 16):
            sb[0, pl.ds(l * 16, 16)] = eb[0, pl.ds(l * 16, 16)]
            db[0, pl.ds(l * 16, 16)] = eb[1, pl.ds(l * 16, 16)]

    @pl.when((c == 0) & (s < NS - 1))
    def _():
        pltpu.sync_copy(xw_hbm.at[pl.ds(row0, ROWS_PT)],
                        acc_sh.at[pl.ds(row0, ROWS_PT)])

    @pl.when((c == 1) | (s == NS - 1))
    def _():
        def zb(i, carry):
            for l in range(HID // 16):
                buf0[i, pl.ds(l * 16, 16)] = jnp.zeros((16,), jnp.float32)
            return carry
        lax.fori_loop(0, CHUNK, zb, 0)

        @pl.when(c == 1)
        def _():
            def cp(i, carry):
                pltpu.sync_copy(buf0,
                                acc_sh.at[pl.ds(row0 + i * CHUNK, CHUNK)])
                return carry
            lax.fori_loop(0, ROWS_PT // CHUNK, cp, 0)

        @pl.when(c == 0)
        def _():
            pltpu.sync_copy(xw_hbm.at[pl.ds(row0, N_NODES - (NS - 1) * ROWS_PT)],
                            acc_sh.at[pl.ds(row0, N_NODES - (NS - 1) * ROWS_PT)])
            pltpu.sync_copy(buf0, acc_sh.at[pl.ds(N_NODES, CHUNK)])
            pltpu.sync_copy(buf0.at[pl.ds(0, N_PAD - N_NODES - CHUNK)],
                            acc_sh.at[pl.ds(N_NODES + CHUNK,
                                            N_PAD - N_NODES - CHUNK)])
    plsc.subcore_barrier()

    pltpu.async_copy(esl(0), ei0, semi0)
    pltpu.make_async_copy(esl(0), ei0, semi0).wait()
    unpack(ei0, sidx0, didx0)
    pltpu.async_copy(xw_hbm.at[sidx0.at[0]], buf0, sem0)
    pltpu.async_copy(esl(1), ei1, semi1)

    def pair(p, carry):
        j0 = 2 * p
        pltpu.async_copy(esl(j0 + 2), ei0, semi0)
        pltpu.make_async_copy(esl(0), ei1, semi1).wait()
        unpack(ei1, sidx1, didx1)
        pltpu.async_copy(xw_hbm.at[sidx1.at[0]], buf1, sem1)
        pltpu.make_async_copy(xw_hbm.at[sidx0.at[0]], buf0, sem0).wait()
        pltpu.sync_copy(buf0, acc_sh.at[didx0.at[0]], add=True)
        pltpu.async_copy(esl(j0 + 3), ei1, semi1)
        pltpu.make_async_copy(esl(0), ei0, semi0).wait()
        unpack(ei0, sidx0, didx0)
        pltpu.async_copy(xw_hbm.at[sidx0.at[0]], buf0, sem0)
        pltpu.make_async_copy(xw_hbm.at[sidx1.at[0]], buf1, sem1).wait()
        pltpu.sync_copy(buf1, acc_sh.at[didx1.at[0]], add=True)
        return carry

    lax.fori_loop(0, NPAIR, pair, 0)
    pltpu.make_async_copy(esl(0), ei1, semi1).wait()
    unpack(ei1, sidx1, didx1)
    pltpu.async_copy(xw_hbm.at[sidx1.at[0]], buf1, sem1)
    pltpu.make_async_copy(xw_hbm.at[sidx0.at[0]], buf0, sem0).wait()
    pltpu.sync_copy(buf0, acc_sh.at[didx0.at[0]], add=True)

    @pl.when(nch == NCHB + 1)
    def _():
        pltpu.async_copy(esl(NCHB), ei0, semi0)
    pltpu.make_async_copy(xw_hbm.at[sidx1.at[0]], buf1, sem1).wait()
    pltpu.sync_copy(buf1, acc_sh.at[didx1.at[0]], add=True)

    @pl.when(nch == NCHB + 1)
    def _():
        pltpu.make_async_copy(esl(0), ei0, semi0).wait()
        unpack(ei0, sidx0, didx0)
        pltpu.async_copy(xw_hbm.at[sidx0.at[0]], buf0, sem0)
        pltpu.make_async_copy(xw_hbm.at[sidx0.at[0]], buf0, sem0).wait()
        pltpu.sync_copy(buf0, acc_sh.at[didx0.at[0]], add=True)

    plsc.subcore_barrier()

    @pl.when(c == 0)
    def _():
        pltpu.sync_copy(acc_sh.at[pl.ds(row0, ROWS_PT)],
                        acc0_out.at[pl.ds(row0, ROWS_PT)])

    @pl.when(c == 1)
    def _():
        pltpu.sync_copy(acc_sh.at[pl.ds(row0, ROWS_PT)],
                        acc1_out.at[pl.ds(row0, ROWS_PT)])


_BLKB = 1000
_GRIDB = N_NODES // _BLKB
_BLKD = 1000
_GRIDD = N_NODES // _BLKD


def _mm_body(x_ref, w_ref, out_ref):
    out_ref[...] = jnp.dot(x_ref[...], w_ref[...],
                           preferred_element_type=jnp.float32)


def _run_mm(x, w):
    return pl.pallas_call(
        _mm_body,
        grid=(_GRIDB,),
        in_specs=[
            pl.BlockSpec((_BLKB, D_FEAT), lambda i: (i, 0)),
            pl.BlockSpec((D_FEAT, HID), lambda i: (0, 0)),
        ],
        out_specs=pl.BlockSpec((_BLKB, HID), lambda i: (i, 0)),
        out_shape=jax.ShapeDtypeStruct((N_NODES, HID), jnp.float32),
    )(x, w)


def _scale_body(xw_ref, deg0_ref, deg1_ref, out_ref, dinv_ref):
    dinv = lax.rsqrt(deg0_ref[...] + deg1_ref[...] + 1.0)
    out_ref[...] = xw_ref[...] * dinv
    dinv_ref[...] = dinv


def _run_scale(xw_raw, deg0, deg1):
    return pl.pallas_call(
        _scale_body,
        grid=(_GRIDB,),
        in_specs=[
            pl.BlockSpec((_BLKB, HID), lambda i: (i, 0)),
            pl.BlockSpec((_BLKB, 1), lambda i: (i, 0)),
            pl.BlockSpec((_BLKB, 1), lambda i: (i, 0)),
        ],
        out_specs=[
            pl.BlockSpec((_BLKB, HID), lambda i: (i, 0)),
            pl.BlockSpec((_BLKB, 1), lambda i: (i, 0)),
        ],
        out_shape=[
            jax.ShapeDtypeStruct((N_NODES, HID), jnp.float32),
            jax.ShapeDtypeStruct((N_NODES, 1), jnp.float32),
        ],
    )(xw_raw, deg0, deg1)


def _head_body(acc0_ref, acc1_ref, dinv_ref, h_ref, bgcn_ref,
               wiht_ref, whht_ref, bih_ref, bhh_ref, wfc2t_ref, bfc2_ref,
               q_ref, hout_ref):
    dinv = dinv_ref[...]
    x = jnp.maximum(
        (acc0_ref[...] + acc1_ref[...]) * dinv + bgcn_ref[...], 0.0)
    h = h_ref[...]
    gi = jnp.dot(x, wiht_ref[...], preferred_element_type=jnp.float32) \
        + bih_ref[...]
    gh = jnp.dot(h, whht_ref[...], preferred_element_type=jnp.float32) \
        + bhh_ref[...]
    r = jax.nn.sigmoid(gi[:, :HID] + gh[:, :HID])
    z = jax.nn.sigmoid(gi[:, HID:2 * HID] + gh[:, HID:2 * HID])
    n = jnp.tanh(gi[:, 2 * HID:] + r * gh[:, 2 * HID:])
    hn = (1.0 - z) * n + z * h
    hout_ref[...] = hn
    q_ref[...] = jnp.dot(hn, wfc2t_ref[...],
                         preferred_element_type=jnp.float32) + bfc2_ref[...]


def _run_head(acc0, acc1, dinv, h, bgcn, wiht, whht, bih, bhh,
              wfc2t, bfc2):
    return pl.pallas_call(
        _head_body,
        grid=(_GRIDD,),
        in_specs=[
            pl.BlockSpec((_BLKD, HID), lambda i: (i, 0)),
            pl.BlockSpec((_BLKD, HID), lambda i: (i, 0)),
            pl.BlockSpec((_BLKD, 1), lambda i: (i, 0)),
            pl.BlockSpec((_BLKD, HID), lambda i: (i, 0)),
            pl.BlockSpec((1, HID), lambda i: (0, 0)),
            pl.BlockSpec((HID, 3 * HID), lambda i: (0, 0)),
            pl.BlockSpec((HID, 3 * HID), lambda i: (0, 0)),
            pl.BlockSpec((1, 3 * HID), lambda i: (0, 0)),
            pl.BlockSpec((1, 3 * HID), lambda i: (0, 0)),
            pl.BlockSpec((HID, ACT), lambda i: (0, 0)),
            pl.BlockSpec((1, ACT), lambda i: (0, 0)),
        ],
        out_specs=[
            pl.BlockSpec((_BLKD, ACT), lambda i: (i, 0)),
            pl.BlockSpec((_BLKD, HID), lambda i: (i, 0)),
        ],
        out_shape=[
            jax.ShapeDtypeStruct((N_NODES, ACT), jnp.float32),
            jax.ShapeDtypeStruct((N_NODES, HID), jnp.float32),
        ],
    )(acc0, acc1, dinv, h, bgcn, wiht, whht, bih, bhh, wfc2t, bfc2)


def kernel(inputs, hidden_state, edge_index, W_gcn, b_gcn,
           W_ih, W_hh, b_ih, b_hh, W_fc2, b_fc2):
    deg_parts = _deg_kernel(edge_index)
    xw_raw = _run_mm(inputs, W_gcn)
    deg0 = deg_parts[0, :N_NODES].reshape(N_NODES, 1)
    deg1 = deg_parts[1, :N_NODES].reshape(N_NODES, 1)
    xw, dinv = _run_scale(xw_raw, deg0, deg1)

    acc0, acc1 = _msg_kernel(xw, edge_index)

    q, h_new = _run_head(
        acc0, acc1, dinv, hidden_state.reshape(N_NODES, HID),
        b_gcn.reshape(1, HID), W_ih.T, W_hh.T,
        b_ih.reshape(1, 3 * HID), b_hh.reshape(1, 3 * HID),
        W_fc2.T, b_fc2.reshape(1, ACT))
    return (q, h_new)


# speedup vs baseline: 1.2630x; 1.0584x over previous
import functools

import jax
import jax.numpy as jnp
from jax import lax
from jax.experimental import pallas as pl
from jax.experimental.pallas import tpu as pltpu
from jax.experimental.pallas import tpu_sc as plsc

N_NODES = 10000
N_PAD = 10240
E_TOT = 320000
D_FEAT = 128
HID = 128
ACT = 16
NC, NS = 2, 16
NW = NC * NS
CHUNK = 128
NCHT = E_TOT // CHUNK
NCHB = NCHT // NW
NEXTRA = NCHT - NCHB * NW
NPAIR = NCHB // 2 - 1
ROWS_PT = N_PAD // NS

_mesh = plsc.VectorSubcoreMesh(core_axis_name="c", subcore_axis_name="s")


@functools.partial(
    pl.kernel,
    out_type=jax.ShapeDtypeStruct((NC, N_PAD), jnp.float32),
    mesh=_mesh,
    scratch_types=[
        pltpu.VMEM((2, CHUNK), jnp.int32),
        pltpu.VMEM((2, CHUNK), jnp.int32),
        pltpu.VMEM((2, CHUNK), jnp.int32),
        pltpu.VMEM((2, CHUNK), jnp.int32),
        pltpu.VMEM((1, CHUNK), jnp.int32),
        pltpu.VMEM((CHUNK,), jnp.float32),
        pltpu.VMEM((ROWS_PT,), jnp.float32),
        pltpu.VMEM_SHARED((N_PAD,), jnp.float32),
        pltpu.SemaphoreType.DMA,
        pltpu.SemaphoreType.DMA,
        pltpu.SemaphoreType.DMA,
        pltpu.SemaphoreType.DMA,
    ],
)
def _deg_kernel(edge_hbm, deg_out, di0, di1, di2, di3, didx, ones_v,
                zero_v, deg_sh, sm0, sm1, sm2, sm3):
    c = lax.axis_index("c")
    s = lax.axis_index("s")
    wid = s * NC + c
    start = NCHB * wid + jnp.minimum(wid, NEXTRA)
    nch = NCHB + jnp.where(wid < NEXTRA, 1, 0)

    def esl(j, n):
        return edge_hbm.at[pl.ds(0, 2), pl.ds((start + j) * CHUNK, n)]

    def to_didx(eb):
        for l in range(CHUNK // 16):
            didx[0, pl.ds(l * 16, 16)] = eb[1, pl.ds(l * 16, 16)]

    for i in range(CHUNK // 16):
        ones_v[pl.ds(i * 16, 16)] = jnp.ones((16,), jnp.float32)
    for i in range(ROWS_PT // 16):
        zero_v[pl.ds(i * 16, 16)] = jnp.zeros((16,), jnp.float32)
    pltpu.sync_copy(zero_v, deg_sh.at[pl.ds(s * ROWS_PT, ROWS_PT)])
    plsc.subcore_barrier()

    bufs = (di0, di1, di2, di3)
    sems = (sm0, sm1, sm2, sm3)
    for k in range(4):
        pltpu.async_copy(esl(k, CHUNK), bufs[k], sems[k])

    def body(q, carry):
        for k in range(4):
            j = 4 * q + k
            pltpu.make_async_copy(esl(0, CHUNK), bufs[k], sems[k]).wait()
            to_didx(bufs[k])
            pltpu.sync_copy(ones_v, deg_sh.at[didx.at[0]], add=True)

            @pl.when(j + 4 < nch)
            def _():
                pltpu.async_copy(esl(j + 4, CHUNK), bufs[k], sems[k])
        return carry

    lax.fori_loop(0, NCHB // 4, body, 0)
    for k in range(2):
        pltpu.make_async_copy(esl(0, CHUNK), bufs[k], sems[k]).wait()
        to_didx(bufs[k])
        pltpu.sync_copy(ones_v, deg_sh.at[didx.at[0]], add=True)

    @pl.when(nch == NCHB + 1)
    def _():
        pltpu.make_async_copy(esl(0, CHUNK), bufs[2], sems[2]).wait()
        to_didx(bufs[2])
        pltpu.sync_copy(ones_v, deg_sh.at[didx.at[0]], add=True)

    plsc.subcore_barrier()
    pltpu.sync_copy(deg_sh.at[pl.ds(s * ROWS_PT, ROWS_PT)],
                    deg_out.at[c, pl.ds(s * ROWS_PT, ROWS_PT)])


@functools.partial(
    pl.kernel,
    out_type=[
        jax.ShapeDtypeStruct((N_PAD, HID), jnp.float32),
        jax.ShapeDtypeStruct((N_PAD, HID), jnp.float32),
    ],
    mesh=_mesh,
    scratch_types=[
        pltpu.VMEM((2, CHUNK), jnp.int32),
        pltpu.VMEM((2, CHUNK), jnp.int32),
        pltpu.VMEM((1, CHUNK), jnp.int32),
        pltpu.VMEM((1, CHUNK), jnp.int32),
        pltpu.VMEM((1, CHUNK), jnp.int32),
        pltpu.VMEM((1, CHUNK), jnp.int32),
        pltpu.VMEM((CHUNK, HID), jnp.float32),
        pltpu.VMEM((CHUNK, HID), jnp.float32),
        pltpu.VMEM_SHARED((N_PAD, HID), jnp.float32),
        pltpu.SemaphoreType.DMA,
        pltpu.SemaphoreType.DMA,
        pltpu.SemaphoreType.DMA,
        pltpu.SemaphoreType.DMA,
    ],
)
def _msg_kernel(xw_hbm, edge_hbm, acc0_out, acc1_out,
                ei0, ei1, sidx0, sidx1, didx0, didx1, buf0, buf1, acc_sh,
                sem0, sem1, semi0, semi1):
    c = lax.axis_index("c")
    s = lax.axis_index("s")
    wid = s * NC + c
    start = NCHB * wid + jnp.minimum(wid, NEXTRA)
    nch = NCHB + jnp.where(wid < NEXTRA, 1, 0)
    row0 = s * ROWS_PT

    def esl(j):
        return edge_hbm.at[pl.ds(0, 2), pl.ds((start + j) * CHUNK, CHUNK)]

    def unpack(eb, sb, db):
        for l in range(CHUNK // 16):
            sb[0, pl.ds(l * 16, 16)] = eb[0, pl.ds(l * 16, 16)]
            db[0, pl.ds(l * 16, 16)] = eb[1, pl.ds(l * 16, 16)]

    @pl.when(c == 0)
    def _():
        pltpu.sync_copy(xw_hbm.at[pl.ds(row0, ROWS_PT)],
                        acc_sh.at[pl.ds(row0, ROWS_PT)])

    @pl.when(c == 1)
    def _():
        def zb(i, carry):
            for l in range(HID // 16):
                buf0[i, pl.ds(l * 16, 16)] = jnp.zeros((16,), jnp.float32)
            return carry
        lax.fori_loop(0, CHUNK, zb, 0)

        def cp(i, carry):
            pltpu.sync_copy(buf0, acc_sh.at[pl.ds(row0 + i * CHUNK, CHUNK)])
            return carry
        lax.fori_loop(0, ROWS_PT // CHUNK, cp, 0)
    plsc.subcore_barrier()

    pltpu.async_copy(esl(0), ei0, semi0)
    pltpu.make_async_copy(esl(0), ei0, semi0).wait()
    unpack(ei0, sidx0, didx0)
    pltpu.async_copy(xw_hbm.at[sidx0.at[0]], buf0, sem0)
    pltpu.async_copy(esl(1), ei1, semi1)

    def pair(p, carry):
        j0 = 2 * p
        pltpu.async_copy(esl(j0 + 2), ei0, semi0)
        pltpu.make_async_copy(esl(0), ei1, semi1).wait()
        unpack(ei1, sidx1, didx1)
        pltpu.async_copy(xw_hbm.at[sidx1.at[0]], buf1, sem1)
        pltpu.make_async_copy(xw_hbm.at[sidx0.at[0]], buf0, sem0).wait()
        pltpu.sync_copy(buf0, acc_sh.at[didx0.at[0]], add=True)
        pltpu.async_copy(esl(j0 + 3), ei1, semi1)
        pltpu.make_async_copy(esl(0), ei0, semi0).wait()
        unpack(ei0, sidx0, didx0)
        pltpu.async_copy(xw_hbm.at[sidx0.at[0]], buf0, sem0)
        pltpu.make_async_copy(xw_hbm.at[sidx1.at[0]], buf1, sem1).wait()
        pltpu.sync_copy(buf1, acc_sh.at[didx1.at[0]], add=True)
        return carry

    lax.fori_loop(0, NPAIR, pair, 0)
    pltpu.make_async_copy(esl(0), ei1, semi1).wait()
    unpack(ei1, sidx1, didx1)
    pltpu.async_copy(xw_hbm.at[sidx1.at[0]], buf1, sem1)
    pltpu.make_async_copy(xw_hbm.at[sidx0.at[0]], buf0, sem0).wait()
    pltpu.sync_copy(buf0, acc_sh.at[didx0.at[0]], add=True)

    @pl.when(nch == NCHB + 1)
    def _():
        pltpu.async_copy(esl(NCHB), ei0, semi0)
    pltpu.make_async_copy(xw_hbm.at[sidx1.at[0]], buf1, sem1).wait()
    pltpu.sync_copy(buf1, acc_sh.at[didx1.at[0]], add=True)

    @pl.when(nch == NCHB + 1)
    def _():
        pltpu.make_async_copy(esl(0), ei0, semi0).wait()
        unpack(ei0, sidx0, didx0)
        pltpu.async_copy(xw_hbm.at[sidx0.at[0]], buf0, sem0)
        pltpu.make_async_copy(xw_hbm.at[sidx0.at[0]], buf0, sem0).wait()
        pltpu.sync_copy(buf0, acc_sh.at[didx0.at[0]], add=True)

    plsc.subcore_barrier()

    @pl.when(c == 0)
    def _():
        pltpu.sync_copy(acc_sh.at[pl.ds(row0, ROWS_PT)],
                        acc0_out.at[pl.ds(row0, ROWS_PT)])

    @pl.when(c == 1)
    def _():
        pltpu.sync_copy(acc_sh.at[pl.ds(row0, ROWS_PT)],
                        acc1_out.at[pl.ds(row0, ROWS_PT)])


_BLKB = 2048
_GRIDB = N_PAD // _BLKB
_BLKD = 1000
_GRIDD = N_NODES // _BLKD


def _mm_body(x_ref, w_ref, out_ref):
    out_ref[...] = jnp.dot(x_ref[...], w_ref[...],
                           preferred_element_type=jnp.float32)


def _run_mm(x_pad, w):
    return pl.pallas_call(
        _mm_body,
        grid=(_GRIDB,),
        in_specs=[
            pl.BlockSpec((_BLKB, D_FEAT), lambda i: (i, 0)),
            pl.BlockSpec((D_FEAT, HID), lambda i: (0, 0)),
        ],
        out_specs=pl.BlockSpec((_BLKB, HID), lambda i: (i, 0)),
        out_shape=jax.ShapeDtypeStruct((N_PAD, HID), jnp.float32),
    )(x_pad, w)


def _scale_body(xw_ref, deg_ref, out_ref, dinv_ref):
    deg = deg_ref[...]
    dinv = lax.rsqrt(deg[0, :] + deg[1, :] + 1.0)
    out_ref[...] = xw_ref[...] * dinv[:, None]
    dinv_ref[...] = dinv[:, None]


def _run_scale(xw_raw, deg_parts):
    return pl.pallas_call(
        _scale_body,
        grid=(_GRIDB,),
        in_specs=[
            pl.BlockSpec((_BLKB, HID), lambda i: (i, 0)),
            pl.BlockSpec((2, _BLKB), lambda i: (0, i)),
        ],
        out_specs=[
            pl.BlockSpec((_BLKB, HID), lambda i: (i, 0)),
            pl.BlockSpec((_BLKB, 1), lambda i: (i, 0)),
        ],
        out_shape=[
            jax.ShapeDtypeStruct((N_PAD, HID), jnp.float32),
            jax.ShapeDtypeStruct((N_PAD, 1), jnp.float32),
        ],
    )(xw_raw, deg_parts)


def _head_body(acc0_ref, acc1_ref, dinv_ref, h_ref, bgcn_ref,
               wiht_ref, whht_ref, bih_ref, bhh_ref, wfc2t_ref, bfc2_ref,
               q_ref, hout_ref):
    dinv = dinv_ref[...]
    x = jnp.maximum(
        (acc0_ref[...] + acc1_ref[...]) * dinv + bgcn_ref[...], 0.0)
    h = h_ref[...]
    gi = jnp.dot(x, wiht_ref[...], preferred_element_type=jnp.float32) \
        + bih_ref[...]
    gh = jnp.dot(h, whht_ref[...], preferred_element_type=jnp.float32) \
        + bhh_ref[...]
    r = jax.nn.sigmoid(gi[:, :HID] + gh[:, :HID])
    z = jax.nn.sigmoid(gi[:, HID:2 * HID] + gh[:, HID:2 * HID])
    n = jnp.tanh(gi[:, 2 * HID:] + r * gh[:, 2 * HID:])
    hn = (1.0 - z) * n + z * h
    hout_ref[...] = hn
    q_ref[...] = jnp.dot(hn, wfc2t_ref[...],
                         preferred_element_type=jnp.float32) + bfc2_ref[...]


def _run_head(acc0, acc1, dinv, h, bgcn, wiht, whht, bih, bhh,
              wfc2t, bfc2):
    return pl.pallas_call(
        _head_body,
        grid=(_GRIDD,),
        in_specs=[
            pl.BlockSpec((_BLKD, HID), lambda i: (i, 0)),
            pl.BlockSpec((_BLKD, HID), lambda i: (i, 0)),
            pl.BlockSpec((_BLKD, 1), lambda i: (i, 0)),
            pl.BlockSpec((_BLKD, HID), lambda i: (i, 0)),
            pl.BlockSpec((1, HID), lambda i: (0, 0)),
            pl.BlockSpec((HID, 3 * HID), lambda i: (0, 0)),
            pl.BlockSpec((HID, 3 * HID), lambda i: (0, 0)),
            pl.BlockSpec((1, 3 * HID), lambda i: (0, 0)),
            pl.BlockSpec((1, 3 * HID), lambda i: (0, 0)),
            pl.BlockSpec((HID, ACT), lambda i: (0, 0)),
            pl.BlockSpec((1, ACT), lambda i: (0, 0)),
        ],
        out_specs=[
            pl.BlockSpec((_BLKD, ACT), lambda i: (i, 0)),
            pl.BlockSpec((_BLKD, HID), lambda i: (i, 0)),
        ],
        out_shape=[
            jax.ShapeDtypeStruct((N_NODES, ACT), jnp.float32),
            jax.ShapeDtypeStruct((N_NODES, HID), jnp.float32),
        ],
    )(acc0, acc1, dinv, h, bgcn, wiht, whht, bih, bhh, wfc2t, bfc2)


def kernel(inputs, hidden_state, edge_index, W_gcn, b_gcn,
           W_ih, W_hh, b_ih, b_hh, W_fc2, b_fc2):
    deg_parts = _deg_kernel(edge_index)
    x_pad = jnp.pad(inputs, ((0, N_PAD - N_NODES), (0, 0)))
    xw_raw = _run_mm(x_pad, W_gcn)
    xw, dinv = _run_scale(xw_raw, deg_parts)

    acc0, acc1 = _msg_kernel(xw, edge_index)

    q, h_new = _run_head(
        acc0, acc1, dinv, hidden_state.reshape(N_NODES, HID),
        b_gcn.reshape(1, HID), W_ih.T, W_hh.T,
        b_ih.reshape(1, 3 * HID), b_hh.reshape(1, 3 * HID),
        W_fc2.T, b_fc2.reshape(1, ACT))
    return (q, h_new)
